# baseline (speedup 1.0000x reference)
import functools

import jax
import jax.numpy as jnp
from jax.experimental import pallas as pl
from jax.experimental.pallas import tpu as pltpu

D_MODEL_C = 768
N_EXP_C = 8
D_EXP_C = 3072
N_TOK_C = 2048
F_BLK = 768
N_FBLK = D_EXP_C // F_BLK


def _moe_body(x_ref, wr_ref, w1_ref, b1_ref, w2_ref, b2_ref,
              out_ref, aux_ref, wfull_ref, xb_ref):
    e = pl.program_id(0)
    f = pl.program_id(1)
    first = jnp.logical_and(e == 0, f == 0)

    @pl.when(first)
    def _router():
        xf = x_ref[...]
        xb_ref[...] = xf.astype(jnp.bfloat16)
        logits = jax.lax.dot_general(
            xf, wr_ref[...], (((1,), (1,)), ((), ())),
            preferred_element_type=jnp.float32)
        mx = jnp.max(logits, axis=1, keepdims=True)
        ex = jnp.exp(logits - mx)
        se = jnp.sum(ex, axis=1, keepdims=True)
        probs = ex / se
        logz = mx + jnp.log(se)
        z_loss = jnp.sum(logz * logz) / N_TOK_C

        iota = jax.lax.broadcasted_iota(jnp.int32, (N_TOK_C, N_EXP_C), 1)
        m1 = jnp.max(probs, axis=1, keepdims=True)
        i1 = jnp.min(jnp.where(probs == m1, iota, N_EXP_C),
                     axis=1, keepdims=True)
        sel1 = iota == i1
        probs2 = jnp.where(sel1, -1.0, probs)
        m2 = jnp.max(probs2, axis=1, keepdims=True)
        i2 = jnp.min(jnp.where(probs2 == m2, iota, N_EXP_C),
                     axis=1, keepdims=True)
        sel2 = iota == i2
        denom = jnp.maximum(m1 + m2, 1e-9)
        wfull = (m1 / denom) * sel1.astype(jnp.float32) \
            + (m2 / denom) * sel2.astype(jnp.float32)
        wfull_ref[...] = wfull

        mask = sel1.astype(jnp.float32) + sel2.astype(jnp.float32)
        counts = jnp.sum(mask, axis=0, keepdims=True)
        total_sel = jnp.maximum(jnp.sum(counts), 1.0)
        p_mean = jnp.sum(probs, axis=0, keepdims=True) / N_TOK_C
        aux = N_EXP_C * jnp.sum((counts / total_sel) * p_mean)
        aux_ref[...] = (0.01 * aux + 0.001 * z_loss).reshape(1, 1)

    xb = xb_ref[...]
    w1b = w1_ref[0].astype(jnp.bfloat16)
    h = jax.lax.dot_general(
        xb, w1b, (((1,), (1,)), ((), ())),
        preferred_element_type=jnp.float32)
    h = jnp.maximum(h + b1_ref[0], 0.0)
    hb = h.astype(jnp.bfloat16)
    w2b = w2_ref[0].astype(jnp.bfloat16)
    y = jax.lax.dot_general(
        hb, w2b, (((1,), (1,)), ((), ())),
        preferred_element_type=jnp.float32)

    iota = jax.lax.broadcasted_iota(jnp.int32, (N_TOK_C, N_EXP_C), 1)
    wcol = jnp.sum(jnp.where(iota == e, wfull_ref[...], 0.0),
                   axis=1, keepdims=True)

    @pl.when(f == 0)
    def _bias2():
        y_b = y + b2_ref[0]
        contrib = wcol * y_b

        @pl.when(e == 0)
        def _init():
            out_ref[...] = contrib

        @pl.when(e != 0)
        def _acc():
            out_ref[...] += contrib

    @pl.when(f != 0)
    def _nob():
        out_ref[...] += wcol * y


@jax.jit
def kernel(x, W_router, W1, b1, W2, b2):
    out, aux = pl.pallas_call(
        _moe_body,
        grid=(N_EXP_C, N_FBLK),
        in_specs=[
            pl.BlockSpec((N_TOK_C, D_MODEL_C), lambda e, f: (0, 0)),
            pl.BlockSpec((N_EXP_C, D_MODEL_C), lambda e, f: (0, 0)),
            pl.BlockSpec((1, F_BLK, D_MODEL_C), lambda e, f: (e, f, 0)),
            pl.BlockSpec((1, 1, F_BLK), lambda e, f: (e * N_FBLK + f, 0, 0)),
            pl.BlockSpec((1, D_MODEL_C, F_BLK), lambda e, f: (e, 0, f)),
            pl.BlockSpec((1, 1, D_MODEL_C), lambda e, f: (e, 0, 0)),
        ],
        out_specs=[
            pl.BlockSpec((N_TOK_C, D_MODEL_C), lambda e, f: (0, 0)),
            pl.BlockSpec((1, 1), lambda e, f: (0, 0)),
        ],
        out_shape=[
            jax.ShapeDtypeStruct((N_TOK_C, D_MODEL_C), jnp.float32),
            jax.ShapeDtypeStruct((1, 1), jnp.float32),
        ],
        scratch_shapes=[
            pltpu.VMEM((N_TOK_C, N_EXP_C), jnp.float32),
            pltpu.VMEM((N_TOK_C, D_MODEL_C), jnp.bfloat16),
        ],
    )(x, W_router, W1,
      b1.reshape(N_EXP_C * N_FBLK, 1, F_BLK),
      W2,
      b2.reshape(N_EXP_C, 1, D_MODEL_C))
    return out, aux[0, 0]
